# baseline (speedup 1.0000x reference)
import jax
import jax.numpy as jnp
from jax.experimental import pallas as pl
from jax.experimental.pallas import tpu as pltpu
from jax.experimental.pallas import tpu_sc as plsc

_VOCAB = 100000
_EMBED = 64
_HIDDEN = 512
_NPREV = 20
_BATCH = 1024

_GW = 128
_BN = 2048
_PAIR = 2 * _EMBED


def _sc_gather_pairs(table_pairs, pair_idx):
    n = pair_idx.shape[0]
    idx2 = pair_idx.reshape(1, n)
    mesh = plsc.VectorSubcoreMesh(core_axis_name="c", subcore_axis_name="s")

    @pl.kernel(out_type=jax.ShapeDtypeStruct((n, _PAIR), table_pairs.dtype),
               mesh=mesh)
    def gather_kernel(table_hbm, idx_hbm, out_hbm):
        def body(i_vmem, o_vmem):
            pltpu.sync_copy(table_hbm.at[i_vmem.at[0]], o_vmem)

        pltpu.emit_pipeline(
            body,
            grid=(n // _GW,),
            in_specs=[pl.BlockSpec((1, _GW), lambda i: (0, i))],
            out_specs=[pl.BlockSpec((_GW, _PAIR), lambda i: (i, 0))],
            core_axis_name=("c", "s"),
            dimension_semantics=(pltpu.PARALLEL,),
        )(idx_hbm, out_hbm)

    return gather_kernel(table_pairs, idx2)


def _mlp_body(g_ref, idx_ref, w1p_ref, b1_ref, w2_ref, b2_ref, out_ref,
              h_ref):
    j = pl.program_id(0)

    @pl.when(j == 0)
    def _():
        half = jax.lax.broadcasted_iota(
            jnp.int32, (_BATCH, _PAIR), 1) // _EMBED
        acc = jnp.zeros((_BATCH, _HIDDEN), jnp.float32)
        for t in range(_NPREV):
            gt = g_ref[:, t * _PAIR:(t + 1) * _PAIR]
            pt = idx_ref[:, t:t + 1] & 1
            sel = jnp.where(half == pt, gt, 0.0)
            acc = acc + jnp.dot(sel, w1p_ref[t],
                                preferred_element_type=jnp.float32)
        h_ref[...] = jnp.tanh(acc + b1_ref[...])

    out_ref[...] = (
        jnp.dot(h_ref[...], w2_ref[...], preferred_element_type=jnp.float32)
        + b2_ref[...])


def _mlp(g, idx, W1p, b1, W2, b2):
    nblk = pl.cdiv(_VOCAB, _BN)
    return pl.pallas_call(
        _mlp_body,
        grid=(nblk,),
        in_specs=[
            pl.BlockSpec((_BATCH, _NPREV * _PAIR), lambda j: (0, 0)),
            pl.BlockSpec((_BATCH, _NPREV), lambda j: (0, 0)),
            pl.BlockSpec((_NPREV, _PAIR, _HIDDEN), lambda j: (0, 0, 0)),
            pl.BlockSpec((_HIDDEN,), lambda j: (0,)),
            pl.BlockSpec((_HIDDEN, _BN), lambda j: (0, j)),
            pl.BlockSpec((_BN,), lambda j: (j,)),
        ],
        out_specs=pl.BlockSpec((_BATCH, _BN), lambda j: (0, j)),
        out_shape=jax.ShapeDtypeStruct((_BATCH, _VOCAB), jnp.float32),
        scratch_shapes=[pltpu.VMEM((_BATCH, _HIDDEN), jnp.float32)],
    )(g, idx, W1p, b1, W2, b2)


def kernel(inputs, embed_table, W1, b1, W2, b2):
    flat_idx = inputs.reshape(-1)
    table_pairs = embed_table.reshape(_VOCAB // 2, _PAIR)
    gathered = _sc_gather_pairs(table_pairs, flat_idx >> 1)
    g = gathered.reshape(_BATCH, _NPREV * _PAIR)
    W1r = W1.reshape(_NPREV, _EMBED, _HIDDEN)
    W1p = jnp.concatenate([W1r, W1r], axis=1)
    return _mlp(g, inputs, W1p, b1, W2, b2)
